# baseline (speedup 1.0000x reference)
import functools

import jax
import jax.numpy as jnp
from jax import lax
from jax.experimental import pallas as pl
from jax.experimental.pallas import tpu as pltpu
from jax.experimental.pallas import tpu_sc as plsc

N = 10000
E = 160000
D = 256
H = 128

NC = 2
NS = 16
CH = 128

CHB = 128
AGG_CHUNKS = 80
E_AGG = NS * AGG_CHUNKS * CHB
DEG_CHUNKS = 40
E_DEG = NC * NS * DEG_CHUNKS * CH

NPAD = N + 16
TILE_BASE = 624
_ZCH = [(0, 128), (128, 128), (256, 128), (384, 128), (512, 128)]


@functools.lru_cache(maxsize=None)
def _sc_mesh():
  return plsc.VectorSubcoreMesh(core_axis_name="c", subcore_axis_name="s",
                                num_cores=NC, num_subcores=NS)


@functools.lru_cache(maxsize=None)
def _deg_call():
  return pl.kernel(
      _deg_kernel,
      out_type=jax.ShapeDtypeStruct((NC, N, H), jnp.float32),
      mesh=_sc_mesh(),
      scratch_types=[
          pltpu.VMEM((DEG_CHUNKS, CH), jnp.int32),
          pltpu.VMEM((CH, H), jnp.float32),
          pltpu.VMEM_SHARED((NPAD, H), jnp.float32),
          pltpu.SemaphoreType.DMA,
      ],
  )


def _deg_kernel(dst_hbm, ones_hbm, zeros_hbm, out_hbm, idx_v, ones_v,
                deg_sh, sem):
  c = lax.axis_index("c")
  s = lax.axis_index("s")
  pltpu.sync_copy(dst_hbm.at[c, s], idx_v)
  pltpu.sync_copy(zeros_hbm, ones_v)
  base = s * TILE_BASE
  for off, n in _ZCH:
    pltpu.sync_copy(ones_v.at[pl.ds(0, n)], deg_sh.at[pl.ds(base + off, n)])
  pltpu.sync_copy(ones_hbm, ones_v)
  plsc.subcore_barrier()

  def body(j, carry):
    pltpu.async_copy(ones_v, deg_sh.at[idx_v.at[j]], sem, add=True)
    return carry

  lax.fori_loop(0, DEG_CHUNKS, body, 0)

  def drain(j, carry):
    pltpu.make_async_copy(ones_v, deg_sh.at[idx_v.at[0]], sem).wait()
    return carry

  lax.fori_loop(0, DEG_CHUNKS, drain, 0)
  plsc.subcore_barrier()
  for off, n in _ZCH:
    pltpu.sync_copy(deg_sh.at[pl.ds(base + off, n)],
                    out_hbm.at[c, pl.ds(base + off, n)])


@functools.lru_cache(maxsize=None)
def _agg_call():
  return pl.kernel(
      _agg_kernel,
      out_type=jax.ShapeDtypeStruct((NC, N, H), jnp.float32),
      mesh=_sc_mesh(),
      scratch_types=[
          pltpu.VMEM((AGG_CHUNKS, CHB), jnp.int32),
          pltpu.VMEM((16, CHB), jnp.int32),
          pltpu.VMEM((CHB, H), jnp.float32),
          pltpu.VMEM((CHB, H), jnp.float32),
          pltpu.VMEM_SHARED((NPAD, H), jnp.float32),
          pltpu.SemaphoreType.DMA,
          pltpu.SemaphoreType.DMA,
          pltpu.SemaphoreType.DMA,
          pltpu.SemaphoreType.DMA,
          pltpu.SemaphoreType.DMA,
      ],
  )


def _agg_kernel(table_hbm, src_hbm, dst_hbm, zeros_hbm, out_hbm, src_v,
                dwin_v, buf0, buf1, agg_sh, semg0, semg1, sems0, sems1,
                semd):
  c = lax.axis_index("c")
  s = lax.axis_index("s")
  bufs = (buf0, buf1)
  semg = (semg0, semg1)
  sems = (sems0, sems1)
  pltpu.sync_copy(src_hbm.at[c, s], src_v)
  pltpu.sync_copy(zeros_hbm, buf0)
  base = s * TILE_BASE
  for k in range(640 // CHB):
    pltpu.sync_copy(buf0, agg_sh.at[pl.ds(base + k * CHB, CHB)])
  pltpu.sync_copy(dst_hbm.at[s, pl.ds(0, 8)], dwin_v.at[pl.ds(0, 8)])
  plsc.subcore_barrier()

  NW = AGG_CHUNKS // 8

  def chunk(j, w, k, first, last):
    b = k % 2
    if not first:
      pltpu.make_async_copy(bufs[1 - b], agg_sh.at[dwin_v.at[0]],
                            sems[1 - b]).wait()
    if k == 1 and not last:
      pltpu.async_copy(dst_hbm.at[s, pl.ds((w + 1) * 8, 8)],
                       dwin_v.at[pl.ds(((w + 1) % 2) * 8, 8)], semd)
    if not (last and k == 7):
      pltpu.async_copy(table_hbm.at[src_v.at[j + 1, pl.ds(0, 64)]],
                       bufs[1 - b].at[pl.ds(0, 64)], semg[1 - b])
      pltpu.async_copy(table_hbm.at[src_v.at[j + 1, pl.ds(64, 64)]],
                       bufs[1 - b].at[pl.ds(64, 64)], semg[1 - b])
    pltpu.make_async_copy(table_hbm.at[src_v.at[j]], bufs[b],
                          semg[b]).wait()
    pltpu.async_copy(bufs[b], agg_sh.at[dwin_v.at[(w % 2) * 8 + k]],
                     sems[b], add=True)

  pltpu.async_copy(table_hbm.at[src_v.at[0, pl.ds(0, 64)]],
                   buf0.at[pl.ds(0, 64)], semg0)
  pltpu.async_copy(table_hbm.at[src_v.at[0, pl.ds(64, 64)]],
                   buf0.at[pl.ds(64, 64)], semg0)
  for k in range(8):
    chunk(k, 0, k, first=(k == 0), last=False)
  pltpu.make_async_copy(dst_hbm.at[s, pl.ds(0, 8)],
                        dwin_v.at[pl.ds(0, 8)], semd).wait()

  def window(w, carry):
    for k in range(8):
      chunk(w * 8 + k, w, k, first=False, last=False)
    pltpu.make_async_copy(dst_hbm.at[s, pl.ds(0, 8)],
                          dwin_v.at[pl.ds(0, 8)], semd).wait()
    return carry

  lax.fori_loop(1, NW - 1, window, 0)
  for k in range(8):
    chunk((NW - 1) * 8 + k, NW - 1, k, first=False, last=True)
  pltpu.make_async_copy(buf1, agg_sh.at[dwin_v.at[0]], sems1).wait()
  plsc.subcore_barrier()
  for off, n in _ZCH:
    pltpu.sync_copy(agg_sh.at[pl.ds(base + off, n)],
                    out_hbm.at[c, pl.ds(base + off, n)])


_BM = 1000
_NB = N // _BM


def _b1_body(x_ref, w_ref, deg_ref, hp_ref, dinv_ref):
  degb = deg_ref[0, :, 0:1] + deg_ref[1, :, 0:1] + 1.0
  dinv = 1.0 / jnp.sqrt(degb)
  acc = jnp.dot(x_ref[...], w_ref[...], preferred_element_type=jnp.float32)
  hp = acc * dinv
  hp_ref[0] = hp[:, :H]
  hp_ref[1] = hp[:, H:]
  dinv_ref[...] = jnp.broadcast_to(dinv, dinv_ref.shape)


def _b2_body(agg_ref, hp_ref, dinv_ref, w_ref, b_ref, out_ref):
  dinv = dinv_ref[:, 0:1]
  pre = jnp.concatenate([agg_ref[0] + hp_ref[0], agg_ref[1] + hp_ref[1]],
                        axis=1)
  h = jnp.maximum(pre * dinv + b_ref[...], 0.0)
  hp = jnp.dot(h, w_ref[...], preferred_element_type=jnp.float32) * dinv
  out_ref[0] = hp[:, :H]
  out_ref[1] = hp[:, H:]


def _b3_body(agg_ref, hp_ref, dinv_ref, b_ref, out_ref):
  dinv = dinv_ref[:, 0:1]
  pre = jnp.concatenate([agg_ref[0] + hp_ref[0], agg_ref[1] + hp_ref[1]],
                        axis=1)
  out_ref[...] = jnp.maximum(pre * dinv + b_ref[...], 0.0)


_b1_call = pl.pallas_call(
    _b1_body,
    grid=(_NB,),
    in_specs=[
        pl.BlockSpec((_BM, D), lambda i: (i, 0)),
        pl.BlockSpec((D, D), lambda i: (0, 0)),
        pl.BlockSpec((NC, _BM, H), lambda i: (0, i, 0)),
    ],
    out_specs=[
        pl.BlockSpec((NC, _BM, H), lambda i: (0, i, 0)),
        pl.BlockSpec((_BM, 16), lambda i: (i, 0)),
    ],
    out_shape=[
        jax.ShapeDtypeStruct((NC, N, H), jnp.float32),
        jax.ShapeDtypeStruct((N, 16), jnp.float32),
    ],
)

_b2_call = pl.pallas_call(
    _b2_body,
    grid=(_NB,),
    in_specs=[
        pl.BlockSpec((NC, _BM, H), lambda i: (0, i, 0)),
        pl.BlockSpec((NC, _BM, H), lambda i: (0, i, 0)),
        pl.BlockSpec((_BM, 16), lambda i: (i, 0)),
        pl.BlockSpec((D, D), lambda i: (0, 0)),
        pl.BlockSpec((1, D), lambda i: (0, 0)),
    ],
    out_specs=pl.BlockSpec((NC, _BM, H), lambda i: (0, i, 0)),
    out_shape=jax.ShapeDtypeStruct((NC, N, H), jnp.float32),
)

_b3_call = pl.pallas_call(
    _b3_body,
    grid=(_NB,),
    in_specs=[
        pl.BlockSpec((NC, _BM, H), lambda i: (0, i, 0)),
        pl.BlockSpec((NC, _BM, H), lambda i: (0, i, 0)),
        pl.BlockSpec((_BM, 16), lambda i: (i, 0)),
        pl.BlockSpec((1, D), lambda i: (0, 0)),
    ],
    out_specs=pl.BlockSpec((_BM, D), lambda i: (i, 0)),
    out_shape=jax.ShapeDtypeStruct((N, D), jnp.float32),
)


def kernel(x, edge_index, W1, b1, W2, b2):
  src = edge_index[0]
  dst = edge_index[1]

  pad = E_AGG - E
  src_p = jnp.concatenate([src, jnp.zeros((pad,), jnp.int32)])
  dst_p = jnp.concatenate([dst, jnp.full((pad,), N, jnp.int32)])
  src_agg = jnp.stack([src_p, src_p + N]).reshape(NC, NS, AGG_CHUNKS, CHB)
  dst_agg = dst_p.reshape(NS, AGG_CHUNKS, CHB)
  dst_deg = dst_p.reshape(NC, NS, DEG_CHUNKS, CH)

  onesH = jnp.ones((CH, H), jnp.float32)
  zerosH = jnp.zeros((CHB, H), jnp.float32)
  b1_2d = b1.reshape(1, D)
  b2_2d = b2.reshape(1, D)

  deg2 = _deg_call()(dst_deg, onesH, zerosH)

  hp1, dinv = _b1_call(x, W1, deg2)
  agg1 = _agg_call()(hp1.reshape(NC * N, H), src_agg, dst_agg, zerosH)
  hp2 = _b2_call(agg1, hp1, dinv, W2, b1_2d)
  agg2 = _agg_call()(hp2.reshape(NC * N, H), src_agg, dst_agg, zerosH)
  return _b3_call(agg2, hp2, dinv, b2_2d)


# speedup vs baseline: 1.0329x; 1.0329x over previous
import functools

import jax
import jax.numpy as jnp
from jax import lax
from jax.experimental import pallas as pl
from jax.experimental.pallas import tpu as pltpu
from jax.experimental.pallas import tpu_sc as plsc

N = 10000
E = 160000
D = 256
H = 128

NC = 2
NS = 16
CH = 128

CHB = 128
AGG_CHUNKS = 80
E_AGG = NS * AGG_CHUNKS * CHB
DEG_CHUNKS = 40
E_DEG = NC * NS * DEG_CHUNKS * CH

NPAD = N + 16
TILE_BASE = 624
_ZCH = [(0, 128), (128, 128), (256, 128), (384, 128), (512, 128)]


@functools.lru_cache(maxsize=None)
def _sc_mesh():
  return plsc.VectorSubcoreMesh(core_axis_name="c", subcore_axis_name="s",
                                num_cores=NC, num_subcores=NS)


@functools.lru_cache(maxsize=None)
def _deg_call():
  return pl.kernel(
      _deg_kernel,
      out_type=jax.ShapeDtypeStruct((NC, N, H), jnp.float32),
      mesh=_sc_mesh(),
      scratch_types=[
          pltpu.VMEM((DEG_CHUNKS, CH), jnp.int32),
          pltpu.VMEM((CH, H), jnp.float32),
          pltpu.VMEM_SHARED((NPAD, H), jnp.float32),
          pltpu.SemaphoreType.DMA,
      ],
  )


def _deg_kernel(dst_hbm, ones_hbm, zeros_hbm, out_hbm, idx_v, ones_v,
                deg_sh, sem):
  c = lax.axis_index("c")
  s = lax.axis_index("s")
  pltpu.sync_copy(dst_hbm.at[c, s], idx_v)
  pltpu.sync_copy(zeros_hbm, ones_v)
  base = s * TILE_BASE
  for off, n in _ZCH:
    pltpu.sync_copy(ones_v.at[pl.ds(0, n)], deg_sh.at[pl.ds(base + off, n)])
  pltpu.sync_copy(ones_hbm, ones_v)
  plsc.subcore_barrier()

  def body(j, carry):
    pltpu.async_copy(ones_v, deg_sh.at[idx_v.at[j]], sem, add=True)
    return carry

  lax.fori_loop(0, DEG_CHUNKS, body, 0)

  def drain(j, carry):
    pltpu.make_async_copy(ones_v, deg_sh.at[idx_v.at[0]], sem).wait()
    return carry

  lax.fori_loop(0, DEG_CHUNKS, drain, 0)
  plsc.subcore_barrier()
  for off, n in _ZCH:
    pltpu.sync_copy(deg_sh.at[pl.ds(base + off, n)],
                    out_hbm.at[c, pl.ds(base + off, n)])


@functools.lru_cache(maxsize=None)
def _agg_call():
  return pl.kernel(
      _agg_kernel,
      out_type=jax.ShapeDtypeStruct((NC, N, H), jnp.float32),
      mesh=_sc_mesh(),
      scratch_types=[
          pltpu.VMEM((AGG_CHUNKS, CHB), jnp.int32),
          pltpu.VMEM((16, CHB), jnp.int32),
          pltpu.VMEM((CHB, H), jnp.float32),
          pltpu.VMEM((CHB, H), jnp.float32),
          pltpu.VMEM_SHARED((NPAD, H), jnp.float32),
          pltpu.SemaphoreType.DMA,
          pltpu.SemaphoreType.DMA,
          pltpu.SemaphoreType.DMA,
          pltpu.SemaphoreType.DMA,
          pltpu.SemaphoreType.DMA,
      ],
  )


def _agg_kernel(table_hbm, src_hbm, dst_hbm, zeros_hbm, out_hbm, src_v,
                dwin_v, buf0, buf1, agg_sh, semg0, semg1, sems0, sems1,
                semd):
  c = lax.axis_index("c")
  s = lax.axis_index("s")
  bufs = (buf0, buf1)
  semg = (semg0, semg1)
  sems = (sems0, sems1)
  pltpu.sync_copy(src_hbm.at[c, s], src_v)
  pltpu.sync_copy(zeros_hbm, buf0)
  base = s * TILE_BASE
  for k in range(640 // CHB):
    pltpu.sync_copy(buf0, agg_sh.at[pl.ds(base + k * CHB, CHB)])
  pltpu.sync_copy(dst_hbm.at[s, pl.ds(0, 8)], dwin_v.at[pl.ds(0, 8)])
  plsc.subcore_barrier()

  NW = AGG_CHUNKS // 8

  def chunk(j, w, k, first, last):
    b = k % 2
    if not first:
      pltpu.make_async_copy(bufs[1 - b], agg_sh.at[dwin_v.at[0]],
                            sems[1 - b]).wait()
    if k == 1 and not last:
      pltpu.async_copy(dst_hbm.at[s, pl.ds((w + 1) * 8, 8)],
                       dwin_v.at[pl.ds(((w + 1) % 2) * 8, 8)], semd)
    if not (last and k == 7):
      pltpu.async_copy(table_hbm.at[src_v.at[j + 1, pl.ds(0, 64)]],
                       bufs[1 - b].at[pl.ds(0, 64)], semg[1 - b])
      pltpu.async_copy(table_hbm.at[src_v.at[j + 1, pl.ds(64, 64)]],
                       bufs[1 - b].at[pl.ds(64, 64)], semg[1 - b])
    pltpu.make_async_copy(table_hbm.at[src_v.at[j]], bufs[b],
                          semg[b]).wait()
    pltpu.async_copy(bufs[b], agg_sh.at[dwin_v.at[(w % 2) * 8 + k]],
                     sems[b], add=True)

  pltpu.async_copy(table_hbm.at[src_v.at[0, pl.ds(0, 64)]],
                   buf0.at[pl.ds(0, 64)], semg0)
  pltpu.async_copy(table_hbm.at[src_v.at[0, pl.ds(64, 64)]],
                   buf0.at[pl.ds(64, 64)], semg0)
  for k in range(8):
    chunk(k, 0, k, first=(k == 0), last=False)
  pltpu.make_async_copy(dst_hbm.at[s, pl.ds(0, 8)],
                        dwin_v.at[pl.ds(0, 8)], semd).wait()

  def window(w, carry):
    for k in range(8):
      chunk(w * 8 + k, w, k, first=False, last=False)
    pltpu.make_async_copy(dst_hbm.at[s, pl.ds(0, 8)],
                          dwin_v.at[pl.ds(0, 8)], semd).wait()
    return carry

  lax.fori_loop(1, NW - 1, window, 0)
  for k in range(8):
    chunk((NW - 1) * 8 + k, NW - 1, k, first=False, last=True)
  pltpu.make_async_copy(buf1, agg_sh.at[dwin_v.at[0]], sems1).wait()
  plsc.subcore_barrier()
  for off, n in _ZCH:
    pltpu.sync_copy(agg_sh.at[pl.ds(base + off, n)],
                    out_hbm.at[c, pl.ds(base + off, n)])


_BM = 1000
_NB = N // _BM


def _bmm_body(x_ref, w_ref, y_ref):
  acc = jnp.dot(x_ref[...], w_ref[...], preferred_element_type=jnp.float32)
  y_ref[0] = acc[:, :H]
  y_ref[1] = acc[:, H:]


def _scale_body(y_ref, deg_ref, hp_ref, dinv_ref):
  degb = deg_ref[0, :, 0:1] + deg_ref[1, :, 0:1] + 1.0
  dinv = 1.0 / jnp.sqrt(degb)
  hp_ref[0] = y_ref[0] * dinv
  hp_ref[1] = y_ref[1] * dinv
  dinv_ref[...] = jnp.broadcast_to(dinv, dinv_ref.shape)


def _b2_body(agg_ref, hp_ref, dinv_ref, w_ref, b_ref, out_ref):
  dinv = dinv_ref[:, 0:1]
  pre = jnp.concatenate([agg_ref[0] + hp_ref[0], agg_ref[1] + hp_ref[1]],
                        axis=1)
  h = jnp.maximum(pre * dinv + b_ref[...], 0.0)
  hp = jnp.dot(h, w_ref[...], preferred_element_type=jnp.float32) * dinv
  out_ref[0] = hp[:, :H]
  out_ref[1] = hp[:, H:]


def _b3_body(agg_ref, hp_ref, dinv_ref, b_ref, out_ref):
  dinv = dinv_ref[:, 0:1]
  pre = jnp.concatenate([agg_ref[0] + hp_ref[0], agg_ref[1] + hp_ref[1]],
                        axis=1)
  out_ref[...] = jnp.maximum(pre * dinv + b_ref[...], 0.0)


_bmm_call = pl.pallas_call(
    _bmm_body,
    grid=(_NB,),
    in_specs=[
        pl.BlockSpec((_BM, D), lambda i: (i, 0)),
        pl.BlockSpec((D, D), lambda i: (0, 0)),
    ],
    out_specs=pl.BlockSpec((NC, _BM, H), lambda i: (0, i, 0)),
    out_shape=jax.ShapeDtypeStruct((NC, N, H), jnp.float32),
)

_scale_call = pl.pallas_call(
    _scale_body,
    grid=(_NB,),
    in_specs=[
        pl.BlockSpec((NC, _BM, H), lambda i: (0, i, 0)),
        pl.BlockSpec((NC, _BM, H), lambda i: (0, i, 0)),
    ],
    out_specs=[
        pl.BlockSpec((NC, _BM, H), lambda i: (0, i, 0)),
        pl.BlockSpec((_BM, 16), lambda i: (i, 0)),
    ],
    out_shape=[
        jax.ShapeDtypeStruct((NC, N, H), jnp.float32),
        jax.ShapeDtypeStruct((N, 16), jnp.float32),
    ],
)

_b2_call = pl.pallas_call(
    _b2_body,
    grid=(_NB,),
    in_specs=[
        pl.BlockSpec((NC, _BM, H), lambda i: (0, i, 0)),
        pl.BlockSpec((NC, _BM, H), lambda i: (0, i, 0)),
        pl.BlockSpec((_BM, 16), lambda i: (i, 0)),
        pl.BlockSpec((D, D), lambda i: (0, 0)),
        pl.BlockSpec((1, D), lambda i: (0, 0)),
    ],
    out_specs=pl.BlockSpec((NC, _BM, H), lambda i: (0, i, 0)),
    out_shape=jax.ShapeDtypeStruct((NC, N, H), jnp.float32),
)

_b3_call = pl.pallas_call(
    _b3_body,
    grid=(_NB,),
    in_specs=[
        pl.BlockSpec((NC, _BM, H), lambda i: (0, i, 0)),
        pl.BlockSpec((NC, _BM, H), lambda i: (0, i, 0)),
        pl.BlockSpec((_BM, 16), lambda i: (i, 0)),
        pl.BlockSpec((1, D), lambda i: (0, 0)),
    ],
    out_specs=pl.BlockSpec((_BM, D), lambda i: (i, 0)),
    out_shape=jax.ShapeDtypeStruct((N, D), jnp.float32),
)


def kernel(x, edge_index, W1, b1, W2, b2):
  src = edge_index[0]
  dst = edge_index[1]

  pad = E_AGG - E
  src_p = jnp.concatenate([src, jnp.zeros((pad,), jnp.int32)])
  dst_p = jnp.concatenate([dst, jnp.full((pad,), N, jnp.int32)])
  src_agg = jnp.stack([src_p, src_p + N]).reshape(NC, NS, AGG_CHUNKS, CHB)
  dst_agg = dst_p.reshape(NS, AGG_CHUNKS, CHB)
  dst_deg = dst_p.reshape(NC, NS, DEG_CHUNKS, CH)

  onesH = jnp.ones((CH, H), jnp.float32)
  zerosH = jnp.zeros((CHB, H), jnp.float32)
  b1_2d = b1.reshape(1, D)
  b2_2d = b2.reshape(1, D)

  deg2 = _deg_call()(dst_deg, onesH, zerosH)
  y1 = _bmm_call(x, W1)
  hp1, dinv = _scale_call(y1, deg2)
  agg1 = _agg_call()(hp1.reshape(NC * N, H), src_agg, dst_agg, zerosH)
  hp2 = _b2_call(agg1, hp1, dinv, W2, b1_2d)
  agg2 = _agg_call()(hp2.reshape(NC * N, H), src_agg, dst_agg, zerosH)
  return _b3_call(agg2, hp2, dinv, b2_2d)


# speedup vs baseline: 1.0355x; 1.0025x over previous
import functools

import jax
import jax.numpy as jnp
from jax import lax
from jax.experimental import pallas as pl
from jax.experimental.pallas import tpu as pltpu
from jax.experimental.pallas import tpu_sc as plsc

N = 10000
E = 160000
D = 256
H = 128

NC = 2
NS = 16
CH = 128

CHB = 128
AGG_CHUNKS = 80
E_AGG = NS * AGG_CHUNKS * CHB
DEG_CHUNKS = 40
E_DEG = NC * NS * DEG_CHUNKS * CH

NPAD = N + 16
TILE_BASE = 624
_ZCH = [(0, 128), (128, 128), (256, 128), (384, 128), (512, 128)]


@functools.lru_cache(maxsize=None)
def _sc_mesh():
  return plsc.VectorSubcoreMesh(core_axis_name="c", subcore_axis_name="s",
                                num_cores=NC, num_subcores=NS)


@functools.lru_cache(maxsize=None)
def _deg_call():
  return pl.kernel(
      _deg_kernel,
      out_type=jax.ShapeDtypeStruct((NC, N, H), jnp.float32),
      mesh=_sc_mesh(),
      scratch_types=[
          pltpu.VMEM((DEG_CHUNKS, CH), jnp.int32),
          pltpu.VMEM((CH, H), jnp.float32),
          pltpu.VMEM_SHARED((NPAD, H), jnp.float32),
          pltpu.SemaphoreType.DMA,
      ],
  )


def _deg_kernel(dst_hbm, ones_hbm, zeros_hbm, out_hbm, idx_v, ones_v,
                deg_sh, sem):
  c = lax.axis_index("c")
  s = lax.axis_index("s")
  pltpu.sync_copy(dst_hbm.at[c, s], idx_v)
  pltpu.sync_copy(zeros_hbm, ones_v)
  base = s * TILE_BASE
  for off, n in _ZCH:
    pltpu.sync_copy(ones_v.at[pl.ds(0, n)], deg_sh.at[pl.ds(base + off, n)])
  pltpu.sync_copy(ones_hbm, ones_v)
  plsc.subcore_barrier()

  def body(j, carry):
    pltpu.async_copy(ones_v, deg_sh.at[idx_v.at[j]], sem, add=True)
    return carry

  lax.fori_loop(0, DEG_CHUNKS, body, 0)

  def drain(j, carry):
    pltpu.make_async_copy(ones_v, deg_sh.at[idx_v.at[0]], sem).wait()
    return carry

  lax.fori_loop(0, DEG_CHUNKS, drain, 0)
  plsc.subcore_barrier()
  for off, n in _ZCH:
    pltpu.sync_copy(deg_sh.at[pl.ds(base + off, n)],
                    out_hbm.at[c, pl.ds(base + off, n)])


@functools.lru_cache(maxsize=None)
def _agg_call():
  return pl.kernel(
      _agg_kernel,
      out_type=jax.ShapeDtypeStruct((NC, N, H), jnp.float32),
      mesh=_sc_mesh(),
      scratch_types=[
          pltpu.VMEM((AGG_CHUNKS, CHB), jnp.int32),
          pltpu.VMEM((16, CHB), jnp.int32),
          pltpu.VMEM((CHB, H), jnp.float32),
          pltpu.VMEM((CHB, H), jnp.float32),
          pltpu.VMEM_SHARED((NPAD, H), jnp.float32),
          pltpu.SemaphoreType.DMA,
          pltpu.SemaphoreType.DMA,
          pltpu.SemaphoreType.DMA,
          pltpu.SemaphoreType.DMA,
          pltpu.SemaphoreType.DMA,
      ],
  )


def _agg_kernel(table_hbm, src_hbm, dst_hbm, zeros_hbm, out_hbm, src_v,
                dwin_v, buf0, buf1, agg_sh, semg0, semg1, sems0, sems1,
                semd):
  c = lax.axis_index("c")
  s = lax.axis_index("s")
  bufs = (buf0, buf1)
  semg = (semg0, semg1)
  sems = (sems0, sems1)
  pltpu.sync_copy(src_hbm.at[c, s], src_v)
  pltpu.sync_copy(zeros_hbm, buf0)
  base = s * TILE_BASE
  for k in range(640 // CHB):
    pltpu.sync_copy(buf0, agg_sh.at[pl.ds(base + k * CHB, CHB)])
  pltpu.sync_copy(dst_hbm.at[s, pl.ds(0, 8)], dwin_v.at[pl.ds(0, 8)])
  pltpu.async_copy(table_hbm.at[src_v.at[0, pl.ds(0, 64)]],
                   buf0.at[pl.ds(0, 64)], semg0)
  pltpu.async_copy(table_hbm.at[src_v.at[0, pl.ds(64, 64)]],
                   buf0.at[pl.ds(64, 64)], semg0)
  pltpu.async_copy(table_hbm.at[src_v.at[1, pl.ds(0, 64)]],
                   buf1.at[pl.ds(0, 64)], semg1)
  pltpu.async_copy(table_hbm.at[src_v.at[1, pl.ds(64, 64)]],
                   buf1.at[pl.ds(64, 64)], semg1)
  plsc.subcore_barrier()

  NW = AGG_CHUNKS // 8

  def chunk(j, w, k, first, last):
    b = k % 2
    if not first:
      pltpu.make_async_copy(bufs[1 - b], agg_sh.at[dwin_v.at[0]],
                            sems[1 - b]).wait()
    if k == 1 and not last:
      pltpu.async_copy(dst_hbm.at[s, pl.ds((w + 1) * 8, 8)],
                       dwin_v.at[pl.ds(((w + 1) % 2) * 8, 8)], semd)
    if not (last and k == 7) and not first:
      pltpu.async_copy(table_hbm.at[src_v.at[j + 1, pl.ds(0, 64)]],
                       bufs[1 - b].at[pl.ds(0, 64)], semg[1 - b])
      pltpu.async_copy(table_hbm.at[src_v.at[j + 1, pl.ds(64, 64)]],
                       bufs[1 - b].at[pl.ds(64, 64)], semg[1 - b])
    pltpu.make_async_copy(table_hbm.at[src_v.at[j]], bufs[b],
                          semg[b]).wait()
    pltpu.async_copy(bufs[b], agg_sh.at[dwin_v.at[(w % 2) * 8 + k]],
                     sems[b], add=True)

  for k in range(8):
    chunk(k, 0, k, first=(k == 0), last=False)
  pltpu.make_async_copy(dst_hbm.at[s, pl.ds(0, 8)],
                        dwin_v.at[pl.ds(0, 8)], semd).wait()

  def window(w, carry):
    for k in range(8):
      chunk(w * 8 + k, w, k, first=False, last=False)
    pltpu.make_async_copy(dst_hbm.at[s, pl.ds(0, 8)],
                          dwin_v.at[pl.ds(0, 8)], semd).wait()
    return carry

  lax.fori_loop(1, NW - 1, window, 0)
  for k in range(8):
    chunk((NW - 1) * 8 + k, NW - 1, k, first=False, last=True)
  pltpu.make_async_copy(buf1, agg_sh.at[dwin_v.at[0]], sems1).wait()
  plsc.subcore_barrier()
  for off, n in _ZCH:
    pltpu.sync_copy(agg_sh.at[pl.ds(base + off, n)],
                    out_hbm.at[c, pl.ds(base + off, n)])


_BM = 1000
_NB = N // _BM


def _bmm_body(x_ref, w_ref, y_ref):
  acc = jnp.dot(x_ref[...], w_ref[...], preferred_element_type=jnp.float32)
  y_ref[0] = acc[:, :H]
  y_ref[1] = acc[:, H:]


def _scale_body(y_ref, deg_ref, hp_ref, dinv_ref):
  degb = deg_ref[0, :, 0:1] + deg_ref[1, :, 0:1] + 1.0
  dinv = 1.0 / jnp.sqrt(degb)
  hp_ref[0] = y_ref[0] * dinv
  hp_ref[1] = y_ref[1] * dinv
  dinv_ref[...] = jnp.broadcast_to(dinv, dinv_ref.shape)


def _b2_body(agg_ref, hp_ref, dinv_ref, w_ref, b_ref, out_ref):
  dinv = dinv_ref[:, 0:1]
  pre = jnp.concatenate([agg_ref[0] + hp_ref[0], agg_ref[1] + hp_ref[1]],
                        axis=1)
  h = jnp.maximum(pre * dinv + b_ref[...], 0.0)
  hp = jnp.dot(h, w_ref[...], preferred_element_type=jnp.float32) * dinv
  out_ref[0] = hp[:, :H]
  out_ref[1] = hp[:, H:]


def _b3_body(agg_ref, hp_ref, dinv_ref, b_ref, out_ref):
  dinv = dinv_ref[:, 0:1]
  pre = jnp.concatenate([agg_ref[0] + hp_ref[0], agg_ref[1] + hp_ref[1]],
                        axis=1)
  out_ref[...] = jnp.maximum(pre * dinv + b_ref[...], 0.0)


_bmm_call = pl.pallas_call(
    _bmm_body,
    grid=(_NB,),
    in_specs=[
        pl.BlockSpec((_BM, D), lambda i: (i, 0)),
        pl.BlockSpec((D, D), lambda i: (0, 0)),
    ],
    out_specs=pl.BlockSpec((NC, _BM, H), lambda i: (0, i, 0)),
    out_shape=jax.ShapeDtypeStruct((NC, N, H), jnp.float32),
)

_scale_call = pl.pallas_call(
    _scale_body,
    grid=(_NB,),
    in_specs=[
        pl.BlockSpec((NC, _BM, H), lambda i: (0, i, 0)),
        pl.BlockSpec((NC, _BM, H), lambda i: (0, i, 0)),
    ],
    out_specs=[
        pl.BlockSpec((NC, _BM, H), lambda i: (0, i, 0)),
        pl.BlockSpec((_BM, 16), lambda i: (i, 0)),
    ],
    out_shape=[
        jax.ShapeDtypeStruct((NC, N, H), jnp.float32),
        jax.ShapeDtypeStruct((N, 16), jnp.float32),
    ],
)

_b2_call = pl.pallas_call(
    _b2_body,
    grid=(_NB,),
    in_specs=[
        pl.BlockSpec((NC, _BM, H), lambda i: (0, i, 0)),
        pl.BlockSpec((NC, _BM, H), lambda i: (0, i, 0)),
        pl.BlockSpec((_BM, 16), lambda i: (i, 0)),
        pl.BlockSpec((D, D), lambda i: (0, 0)),
        pl.BlockSpec((1, D), lambda i: (0, 0)),
    ],
    out_specs=pl.BlockSpec((NC, _BM, H), lambda i: (0, i, 0)),
    out_shape=jax.ShapeDtypeStruct((NC, N, H), jnp.float32),
)

_b3_call = pl.pallas_call(
    _b3_body,
    grid=(_NB,),
    in_specs=[
        pl.BlockSpec((NC, _BM, H), lambda i: (0, i, 0)),
        pl.BlockSpec((NC, _BM, H), lambda i: (0, i, 0)),
        pl.BlockSpec((_BM, 16), lambda i: (i, 0)),
        pl.BlockSpec((1, D), lambda i: (0, 0)),
    ],
    out_specs=pl.BlockSpec((_BM, D), lambda i: (i, 0)),
    out_shape=jax.ShapeDtypeStruct((N, D), jnp.float32),
)


def kernel(x, edge_index, W1, b1, W2, b2):
  src = edge_index[0]
  dst = edge_index[1]

  pad = E_AGG - E
  src_p = jnp.concatenate([src, jnp.zeros((pad,), jnp.int32)])
  dst_p = jnp.concatenate([dst, jnp.full((pad,), N, jnp.int32)])
  src_agg = jnp.stack([src_p, src_p + N]).reshape(NC, NS, AGG_CHUNKS, CHB)
  dst_agg = dst_p.reshape(NS, AGG_CHUNKS, CHB)
  dst_deg = dst_p.reshape(NC, NS, DEG_CHUNKS, CH)

  onesH = jnp.ones((CH, H), jnp.float32)
  zerosH = jnp.zeros((CHB, H), jnp.float32)
  b1_2d = b1.reshape(1, D)
  b2_2d = b2.reshape(1, D)

  deg2 = _deg_call()(dst_deg, onesH, zerosH)
  y1 = _bmm_call(x, W1)
  hp1, dinv = _scale_call(y1, deg2)
  agg1 = _agg_call()(hp1.reshape(NC * N, H), src_agg, dst_agg, zerosH)
  hp2 = _b2_call(agg1, hp1, dinv, W2, b1_2d)
  agg2 = _agg_call()(hp2.reshape(NC * N, H), src_agg, dst_agg, zerosH)
  return _b3_call(agg2, hp2, dinv, b2_2d)
